# Initial kernel scaffold; baseline (speedup 1.0000x reference)
import jax
import jax.numpy as jnp
from jax.experimental import pallas as pl


def kernel(x, edge_index, W1, att_src1, att_dst1, b1, W2, att_src2, att_dst2, b2, ln_g, ln_b, fc_w, fc_b):
    raise NotImplementedError("write your pallas kernel here")


# speedup vs baseline: 16.5080x; 16.5080x over previous
import functools

import jax
import jax.numpy as jnp
from jax import lax
from jax.experimental import pallas as pl
from jax.experimental.pallas import tpu as pltpu
from jax.experimental.pallas import tpu_sc as plsc

N16 = 10240
CHUNK = 128
E_PAD = 331776
BN = 1024


def _edge_pass_builder(D, HS, n_chunks, split_edges):
    assert D == 128
    mesh = plsc.VectorSubcoreMesh(core_axis_name="c", subcore_axis_name="s")
    NH = N16 // 2
    rpt = NH // 16
    CP = 64
    nvr = D // 16
    nmul = 4 * HS
    DR = 2 * N16 // 128

    @functools.partial(
        pl.kernel, mesh=mesh,
        out_type=(jax.ShapeDtypeStruct((2, N16, D), jnp.float32),
                  jax.ShapeDtypeStruct((2, DR, 128), jnp.float32)),
        compiler_params=pltpu.CompilerParams(needs_layout_passes=False),
        scratch_types=[
            pltpu.VMEM((HS * N16,), jnp.float32),
            pltpu.VMEM((HS * N16,), jnp.float32),
            pltpu.VMEM((CHUNK,), jnp.int32),
            pltpu.VMEM((CHUNK,), jnp.int32),
            pltpu.VMEM((CHUNK,), jnp.int32),
            pltpu.VMEM((CHUNK, D), jnp.float32),
            pltpu.VMEM((CHUNK, 128), jnp.float32),
            pltpu.VMEM((HS * CHUNK,), jnp.float32),
            pltpu.VMEM_SHARED((NH + 8, D), jnp.float32),
            pltpu.VMEM_SHARED((DR, 128), jnp.float32),
            pltpu.SemaphoreType.DMA,
        ],
    )
    def k(src_h, dst_h, tab_h, as_h, ad_h, out_h, outd_h,
          abuf_s, abuf_d, sidx, didx, didx2, rows, drows, evals,
          num_sh, den_sh, sem):
        c = lax.axis_index("c")
        w = lax.axis_index("s")
        lane = lax.broadcasted_iota(jnp.int32, (16,), 0)

        for h in range(HS):
            pltpu.sync_copy(as_h.at[c * HS + h], abuf_s.at[pl.ds(h * N16, N16)])
            pltpu.sync_copy(ad_h.at[c * HS + h], abuf_d.at[pl.ds(h * N16, N16)])

        epc = n_chunks * CHUNK
        if split_edges:
            base0 = c * 16 * epc + w * epc
            off = 0
        else:
            base0 = w * epc
            off = c * N16

        for phase in range(2):
            lo = phase * NH

            def zrow(i, _):
                def zcol(r, __):
                    rows[i, pl.ds(r * 16, 16)] = jnp.zeros((16,), jnp.float32)
                    if phase == 0:
                        drows[i, pl.ds(r * 16, 16)] = jnp.zeros(
                            (16,), jnp.float32)
                    return 0
                return lax.fori_loop(0, nvr, zcol, 0)
            lax.fori_loop(0, CHUNK, zrow, 0)

            def zacc(kk, _):
                pltpu.sync_copy(rows.at[pl.ds(0, CP)],
                                num_sh.at[pl.ds(w * rpt + kk * CP, CP)])
                return 0
            lax.fori_loop(0, rpt // CP, zacc, 0)
            if phase == 0:
                @pl.when(w < DR // 16)
                def _zden():
                    pltpu.sync_copy(drows.at[pl.ds(0, 16)],
                                    den_sh.at[pl.ds(w * 16, 16)])
            plsc.subcore_barrier()

            def chunk(g, _):
                base = base0 + g * CHUNK
                pltpu.sync_copy(src_h.at[pl.ds(base, CHUNK)], sidx)
                pltpu.sync_copy(dst_h.at[pl.ds(base, CHUNK)], didx)
                if not split_edges:
                    for q in range(CHUNK // 16):
                        sidx[pl.ds(q * 16, 16)] = sidx[pl.ds(q * 16, 16)] + off
                dma = pltpu.async_copy(tab_h.at[sidx], rows, sem)
                for q in range(CHUNK // 16):
                    sv = sidx[pl.ds(q * 16, 16)] - off
                    dv = didx[pl.ds(q * 16, 16)]
                    for h in range(HS):
                        a = (plsc.load_gather(abuf_s, [sv + h * N16])
                             + plsc.load_gather(abuf_d, [dv + h * N16]))
                        ev = jnp.exp(jnp.where(a > 0, a, a * 0.2))
                        evals[pl.ds(h * CHUNK + q * 16, 16)] = ev
                        if phase == 0:
                            plsc.store_scatter(
                                drows, [lane + q * 16, (dv & 63) * 2 + h], ev)
                    if phase == 0:
                        didx2[pl.ds(q * 16, 16)] = lax.shift_right_logical(dv, 6)
                if phase == 0:
                    pltpu.sync_copy(drows, den_sh.at[didx2], add=True)
                    for q in range(CHUNK // 16):
                        dv = didx[pl.ds(q * 16, 16)]
                        for h in range(HS):
                            plsc.store_scatter(
                                drows, [lane + q * 16, (dv & 63) * 2 + h],
                                jnp.zeros((16,), jnp.float32))
                for q in range(CHUNK // 16):
                    dv = didx[pl.ds(q * 16, 16)]
                    dvp = dv - lo
                    inr = (dvp >= 0) & (dvp < NH)
                    didx[pl.ds(q * 16, 16)] = jnp.where(inr, dvp, NH)

                def medge(j, _):
                    jv = jnp.zeros((16,), jnp.int32) + j
                    spl0 = plsc.load_gather(evals, [jv])
                    if HS == 2:
                        spl1 = plsc.load_gather(evals, [jv + CHUNK])
                    for r in range(nmul):
                        m = spl0 if r < 4 else spl1
                        rows[j, pl.ds(r * 16, 16)] = (
                            rows[j, pl.ds(r * 16, 16)] * m)
                    return 0
                dma.wait()
                lax.fori_loop(0, CHUNK, medge, 0)
                pltpu.sync_copy(rows, num_sh.at[didx], add=True)
                return 0
            lax.fori_loop(0, n_chunks, chunk, 0)
            plsc.subcore_barrier()

            def cpout(kk, _):
                st = w * rpt + kk * CP
                pltpu.sync_copy(num_sh.at[pl.ds(st, CP)], rows.at[pl.ds(0, CP)])
                pltpu.sync_copy(rows.at[pl.ds(0, CP)],
                                out_h.at[c, pl.ds(lo + st, CP)])
                return 0
            lax.fori_loop(0, rpt // CP, cpout, 0)
            if phase == 0:
                @pl.when(w < DR // 16)
                def _cpden():
                    pltpu.sync_copy(den_sh.at[pl.ds(w * 16, 16)],
                                    drows.at[pl.ds(0, 16)])
                    pltpu.sync_copy(drows.at[pl.ds(0, 16)],
                                    outd_h.at[c, pl.ds(w * 16, 16)])
            if phase == 0:
                plsc.subcore_barrier()

    return k


_edge_cache = {}


def _edge_pass(D, HS, n_chunks, split_edges):
    key = (D, HS, n_chunks, split_edges)
    if key not in _edge_cache:
        _edge_cache[key] = _edge_pass_builder(D, HS, n_chunks, split_edges)
    return _edge_cache[key]


def _tc1(x_ref, w_ref, aa_ref, hm_ref, a8_ref):
    hm = jnp.dot(x_ref[...], w_ref[...], preferred_element_type=jnp.float32)
    hm_ref[...] = hm
    a8_ref[...] = jnp.dot(hm, aa_ref[...], preferred_element_type=jnp.float32)


def _tc2(in_ref, dn_ref, b1_ref, w2_ref, aa_ref, hm_ref, a2_ref):
    blk = in_ref[...]
    dn = dn_ref[...]
    outs = []
    for hh in range(4):
        num = blk[:, hh * 64:(hh + 1) * 64]
        den = dn[:, hh:hh + 1]
        outs.append(jnp.maximum(num / den + b1_ref[0, hh * 64:(hh + 1) * 64], 0.0))
    h1 = jnp.concatenate(outs, axis=1)
    hm2 = jnp.dot(h1, w2_ref[...], preferred_element_type=jnp.float32)
    hm_ref[...] = hm2
    a2_ref[...] = jnp.dot(hm2, aa_ref[...], preferred_element_type=jnp.float32)


def _tc3(p0_ref, p1_ref, d0_ref, d1_ref, b2_ref, g_ref, bb_ref, out_ref):
    s = p0_ref[...] + p1_ref[...]
    den = d0_ref[:, 0:1] + d1_ref[:, 0:1]
    h2 = jnp.maximum(s[:, :64] / den + b2_ref[0, :], 0.0)
    mu = jnp.mean(h2, axis=1, keepdims=True)
    var = jnp.mean((h2 - mu) * (h2 - mu), axis=1, keepdims=True)
    out_ref[...] = (h2 - mu) * lax.rsqrt(var + 1e-5) * g_ref[0, :] + bb_ref[0, :]


def _tc4(hs_ref, fw_ref, fb_ref, out_ref):
    blk = hs_ref[...]
    acc = jnp.zeros((blk.shape[0], 64), jnp.float32)
    for t in range(12):
        acc = acc + blk[:, 64 * t: 64 * t + 64]
    hcomb = blk[:, 64 * 11: 64 * 12] + 0.025 * acc
    out_ref[...] = (jnp.dot(hcomb, fw_ref[...], preferred_element_type=jnp.float32)
                    + fb_ref[0, :])


def _rows(n):
    return pl.BlockSpec((BN, n), lambda i: (i, 0))


def _full(r, n):
    return pl.BlockSpec((r, n), lambda i: (0, 0))


def kernel(x, edge_index, W1, att_src1, att_dst1, b1, W2, att_src2, att_dst2,
           b2, ln_g, ln_b, fc_w, fc_b):
    N = x.shape[0]
    G = N16 // BN

    loop = jnp.arange(N, dtype=edge_index.dtype)
    src = jnp.concatenate([edge_index[0], loop]).astype(jnp.int32)
    dst = jnp.concatenate([edge_index[1], loop]).astype(jnp.int32)
    src = jnp.pad(src, (0, E_PAD - src.shape[0]))
    dst = jnp.pad(dst, (0, E_PAD - dst.shape[0]),
                  constant_values=N)
    x_pad = jnp.pad(x, ((0, N16 - N), (0, 0), (0, 0)))
    xs = jnp.moveaxis(x_pad, 1, 0)

    eye4 = jnp.eye(4, dtype=jnp.float32)
    A1 = jnp.concatenate(
        [(eye4[:, None, :] * att_src1[:, :, None]).reshape(256, 4),
         (eye4[:, None, :] * att_dst1[:, :, None]).reshape(256, 4)], axis=1)
    A2 = jnp.concatenate([att_src2.T, att_dst2.T], axis=1)
    zeros64 = jnp.zeros((N16, 64), jnp.float32)
    b1_2d = b1.reshape(1, 256)
    b2_2d = b2.reshape(1, 64)
    g_2d = ln_g.reshape(1, 64)
    bb_2d = ln_b.reshape(1, 64)
    fb_2d = fc_b.reshape(1, 1)

    tck1 = pl.pallas_call(
        _tc1, grid=(G,),
        in_specs=[_rows(128), _full(128, 256), _full(256, 8)],
        out_specs=[_rows(256), _rows(8)],
        out_shape=[jax.ShapeDtypeStruct((N16, 256), jnp.float32),
                   jax.ShapeDtypeStruct((N16, 8), jnp.float32)])
    tck2 = pl.pallas_call(
        _tc2, grid=(G,),
        in_specs=[_rows(256), _rows(4), _full(1, 256), _full(256, 64),
                  _full(64, 2)],
        out_specs=[_rows(64), _rows(2)],
        out_shape=[jax.ShapeDtypeStruct((N16, 64), jnp.float32),
                   jax.ShapeDtypeStruct((N16, 2), jnp.float32)])
    tck3 = pl.pallas_call(
        _tc3, grid=(G,),
        in_specs=[_rows(128), _rows(128), _rows(2), _rows(2),
                  _full(1, 64), _full(1, 64), _full(1, 64)],
        out_specs=_rows(64),
        out_shape=jax.ShapeDtypeStruct((N16, 64), jnp.float32))
    tck4 = pl.pallas_call(
        _tc4, grid=(G,),
        in_specs=[_rows(768), _full(64, 1), _full(1, 1)],
        out_specs=_rows(1),
        out_shape=jax.ShapeDtypeStruct((N16, 1), jnp.float32))

    def step(carry, xt):
        hm, a8 = tck1(xt, W1, A1)
        tab1 = hm.reshape(N16, 2, 128).transpose(1, 0, 2).reshape(2 * N16, 128)
        as1 = a8[:, :4].T
        ad1 = a8[:, 4:].T
        num1, den1p = _edge_pass(128, 2, E_PAD // (16 * CHUNK), False)(
            src, dst, tab1, as1, ad1)
        in2 = num1.transpose(1, 0, 2).reshape(N16, 256)
        den1 = jnp.concatenate(
            [den1p[0].reshape(N16, 2), den1p[1].reshape(N16, 2)], axis=1)
        hm2, a2 = tck2(in2, den1, b1_2d, W2, A2)
        tab2 = jnp.concatenate([hm2, zeros64], axis=1)
        as2 = jnp.tile(a2[:, 0].reshape(1, N16), (2, 1))
        ad2 = jnp.tile(a2[:, 1].reshape(1, N16), (2, 1))
        num2, den2p = _edge_pass(128, 1, E_PAD // (2 * 16 * CHUNK), True)(
            src, dst, tab2, as2, ad2)
        hln = tck3(num2[0], num2[1], den2p[0].reshape(N16, 2),
                   den2p[1].reshape(N16, 2), b2_2d, g_2d, bb_2d)
        return carry, hln

    _, hseq = lax.scan(step, 0, xs)
    hcat = hseq.transpose(1, 0, 2).reshape(N16, 768)
    y = tck4(hcat, fc_w, fb_2d)
    return y[:N, 0]
